# baseline (speedup 1.0000x reference)
import functools

import jax
import jax.numpy as jnp
from jax import lax
from jax.experimental import pallas as pl
from jax.experimental.pallas import tpu as pltpu
from jax.experimental.pallas import tpu_sc as plsc

_L = 16
_NC = 2
_NS = 16
_NW = _NC * _NS

_ROWS = 64
_COLS = 32768
_RB = 8
_NRB = _ROWS // _RB
_WPR = _NW // _NRB
_WCOLS = _COLS // _WPR
_CHUNK = 512
_NCHUNK = _WCOLS // _CHUNK
_NPAIR = _NCHUNK // 2

_mesh = plsc.VectorSubcoreMesh(core_axis_name="c", subcore_axis_name="s")


@functools.partial(
    pl.kernel,
    mesh=_mesh,
    out_type=jax.ShapeDtypeStruct((_ROWS, _COLS), jnp.float32),
    scratch_types=(
        [pltpu.VMEM((_RB, _CHUNK), jnp.float32)] * 8
        + [pltpu.SemaphoreType.DMA] * 4
    ),
)
def _median_sc(a_hbm, b_hbm, c_hbm, out_hbm,
               av0, av1, bv0, bv1, cv0, cv1, ov0, ov1,
               si0, si1, so0, so1):
    wid = lax.axis_index("s") * _NC + lax.axis_index("c")
    r0 = (wid // _WPR) * _RB
    c0 = (wid % _WPR) * _WCOLS

    ins = [(av0, bv0, cv0, si0), (av1, bv1, cv1, si1)]
    outs = [(ov0, so0), (ov1, so1)]

    def hbm_slice(ref, ci):
        return ref.at[pl.ds(r0, _RB), pl.ds(c0 + ci * _CHUNK, _CHUNK)]

    def start_in(ci, p):
        av, bv, cv, si = ins[p]
        pltpu.async_copy(hbm_slice(a_hbm, ci), av, si)
        pltpu.async_copy(hbm_slice(b_hbm, ci), bv, si)
        pltpu.async_copy(hbm_slice(c_hbm, ci), cv, si)

    def wait_in(p):
        av, bv, cv, si = ins[p]
        pltpu.make_async_copy(hbm_slice(a_hbm, 0), av, si).wait()
        pltpu.make_async_copy(hbm_slice(b_hbm, 0), bv, si).wait()
        pltpu.make_async_copy(hbm_slice(c_hbm, 0), cv, si).wait()

    def wait_out(p):
        ov, so = outs[p]
        pltpu.make_async_copy(ov, hbm_slice(out_hbm, 0), so).wait()

    def compute(p, ci):
        av, bv, cv, _ = ins[p]
        ov, so = outs[p]
        @plsc.parallel_loop(0, _RB, 1)
        def row_body(r):
            @plsc.parallel_loop(0, _CHUNK, _L, unroll=8)
            def vec_body(s):
                x = av[r, pl.ds(s, _L)]
                y = bv[r, pl.ds(s, _L)]
                z = cv[r, pl.ds(s, _L)]
                lo = jnp.minimum(x, y)
                hi = jnp.maximum(x, y)
                ov[r, pl.ds(s, _L)] = jnp.maximum(lo, jnp.minimum(hi, z))
        pltpu.async_copy(ov, hbm_slice(out_hbm, ci), so)

    start_in(0, 0)
    start_in(1, 1)

    def pair_body(k, carry):
        even = 2 * k
        wait_in(0)
        lax.cond(k > 0, lambda: wait_out(0), lambda: None)
        compute(0, even)
        lax.cond(k < _NPAIR - 1, lambda: start_in(even + 2, 0), lambda: None)
        wait_in(1)
        lax.cond(k > 0, lambda: wait_out(1), lambda: None)
        compute(1, even + 1)
        lax.cond(k < _NPAIR - 1, lambda: start_in(even + 3, 1), lambda: None)
        return carry

    lax.fori_loop(0, _NPAIR, pair_body, 0)
    wait_out(0)
    wait_out(1)


def kernel(a, b, c):
    return _median_sc(a, b, c)


# speedup vs baseline: 1.0310x; 1.0310x over previous
import functools

import jax
import jax.numpy as jnp
from jax import lax
from jax.experimental import pallas as pl
from jax.experimental.pallas import tpu as pltpu
from jax.experimental.pallas import tpu_sc as plsc

_L = 16
_NC = 2
_NS = 16
_NW = _NC * _NS

_ROWS = 64
_COLS = 32768
_RB = 8
_NRB = _ROWS // _RB
_WPR = _NW // _NRB
_WCOLS = _COLS // _WPR
_CHUNK = 2048
_HALF = _CHUNK // 2
_NCHUNK = _WCOLS // _CHUNK
_NPAIR = _NCHUNK // 2

_mesh = plsc.VectorSubcoreMesh(core_axis_name="c", subcore_axis_name="s")


@functools.partial(
    pl.kernel,
    mesh=_mesh,
    out_type=jax.ShapeDtypeStruct((_ROWS, _COLS), jnp.float32),
    scratch_types=(
        [pltpu.VMEM((_RB, _CHUNK), jnp.float32)] * 6
        + [pltpu.VMEM((_RB, _HALF), jnp.float32)] * 2
        + [pltpu.SemaphoreType.DMA] * 4
    ),
)
def _median_sc(a_hbm, b_hbm, c_hbm, out_hbm,
               av0, av1, bv0, bv1, cv0, cv1, ov0, ov1,
               si0, si1, so0, so1):
    wid = lax.axis_index("s") * _NC + lax.axis_index("c")
    r0 = (wid // _WPR) * _RB
    c0 = (wid % _WPR) * _WCOLS

    ins = [(av0, bv0, cv0, si0), (av1, bv1, cv1, si1)]
    outs = [(ov0, so0), (ov1, so1)]

    def in_slice(ref, ci):
        return ref.at[pl.ds(r0, _RB), pl.ds(c0 + ci * _CHUNK, _CHUNK)]

    def out_slice(hi):
        return out_hbm.at[pl.ds(r0, _RB), pl.ds(c0 + hi * _HALF, _HALF)]

    def start_in(ci, p):
        av, bv, cv, si = ins[p]
        pltpu.async_copy(in_slice(a_hbm, ci), av, si)
        pltpu.async_copy(in_slice(b_hbm, ci), bv, si)
        pltpu.async_copy(in_slice(c_hbm, ci), cv, si)

    def wait_in(p):
        av, bv, cv, si = ins[p]
        pltpu.make_async_copy(in_slice(a_hbm, 0), av, si).wait()
        pltpu.make_async_copy(in_slice(b_hbm, 0), bv, si).wait()
        pltpu.make_async_copy(in_slice(c_hbm, 0), cv, si).wait()

    def wait_out(hp):
        ov, so = outs[hp]
        pltpu.make_async_copy(ov, out_slice(0), so).wait()

    def compute(p, ci, first):
        av, bv, cv, _ = ins[p]
        for h in range(2):
            hp = h
            ov, so = outs[hp]
            if first:
                pass
            else:
                wait_out(hp)

            @plsc.parallel_loop(0, _RB, 1)
            def row_body(r, h=h, av=av, bv=bv, cv=cv, ov=ov):
                @plsc.parallel_loop(0, _HALF, _L, unroll=8)
                def vec_body(s):
                    x = av[r, pl.ds(h * _HALF + s, _L)]
                    y = bv[r, pl.ds(h * _HALF + s, _L)]
                    z = cv[r, pl.ds(h * _HALF + s, _L)]
                    lo = jnp.minimum(x, y)
                    hi2 = jnp.maximum(x, y)
                    ov[r, pl.ds(s, _L)] = jnp.maximum(lo, jnp.minimum(hi2, z))

            pltpu.async_copy(ov, out_slice(2 * ci + h), so)

    start_in(0, 0)
    start_in(1, 1)

    wait_in(0)
    compute(0, 0, first=True)
    start_in(2, 0)
    wait_in(1)
    compute(1, 1, first=False)
    start_in(3, 1)
    wait_in(0)
    compute(0, 2, first=False)
    wait_in(1)
    compute(1, 3, first=False)

    wait_out(0)
    wait_out(1)


def kernel(a, b, c):
    return _median_sc(a, b, c)


# speedup vs baseline: 1.0426x; 1.0113x over previous
import functools

import jax
import jax.numpy as jnp
from jax import lax
from jax.experimental import pallas as pl
from jax.experimental.pallas import tpu as pltpu
from jax.experimental.pallas import tpu_sc as plsc

_L = 16
_NC = 2
_NS = 16
_NW = _NC * _NS

_ROWS = 64
_COLS = 32768
_RB = 8
_NRB = _ROWS // _RB
_WPR = _NW // _NRB
_WCOLS = _COLS // _WPR
_CHUNK = 1024
_NCHUNK = _WCOLS // _CHUNK
_NPAIR = _NCHUNK // 2

_mesh = plsc.VectorSubcoreMesh(core_axis_name="c", subcore_axis_name="s")


@functools.partial(
    pl.kernel,
    mesh=_mesh,
    out_type=jax.ShapeDtypeStruct((_ROWS, _COLS), jnp.float32),
    scratch_types=(
        [pltpu.VMEM((_RB, _CHUNK), jnp.float32)] * 8
        + [pltpu.SemaphoreType.DMA] * 4
    ),
    compiler_params=pltpu.CompilerParams(skip_device_barrier=True),
)
def _median_sc(a_hbm, b_hbm, c_hbm, out_hbm,
               av0, av1, bv0, bv1, cv0, cv1, ov0, ov1,
               si0, si1, so0, so1):
    wid = lax.axis_index("s") * _NC + lax.axis_index("c")
    r0 = (wid // _WPR) * _RB
    c0 = (wid % _WPR) * _WCOLS

    ins = [(av0, bv0, cv0, si0), (av1, bv1, cv1, si1)]
    outs = [(ov0, so0), (ov1, so1)]

    def hbm_slice(ref, ci):
        return ref.at[pl.ds(r0, _RB), pl.ds(c0 + ci * _CHUNK, _CHUNK)]

    def start_in(ci, p):
        av, bv, cv, si = ins[p]
        pltpu.async_copy(hbm_slice(a_hbm, ci), av, si)
        pltpu.async_copy(hbm_slice(b_hbm, ci), bv, si)
        pltpu.async_copy(hbm_slice(c_hbm, ci), cv, si)

    def wait_in(p):
        av, bv, cv, si = ins[p]
        pltpu.make_async_copy(hbm_slice(a_hbm, 0), av, si).wait()
        pltpu.make_async_copy(hbm_slice(b_hbm, 0), bv, si).wait()
        pltpu.make_async_copy(hbm_slice(c_hbm, 0), cv, si).wait()

    def wait_out(p):
        ov, so = outs[p]
        pltpu.make_async_copy(ov, hbm_slice(out_hbm, 0), so).wait()

    def compute(p, ci):
        av, bv, cv, _ = ins[p]
        ov, so = outs[p]

        @plsc.parallel_loop(0, _RB, 1)
        def row_body(r):
            @plsc.parallel_loop(0, _CHUNK, _L, unroll=8)
            def vec_body(s):
                x = av[r, pl.ds(s, _L)]
                y = bv[r, pl.ds(s, _L)]
                z = cv[r, pl.ds(s, _L)]
                lo = jnp.minimum(x, y)
                hi = jnp.maximum(x, y)
                ov[r, pl.ds(s, _L)] = jnp.maximum(lo, jnp.minimum(hi, z))

        pltpu.async_copy(ov, hbm_slice(out_hbm, ci), so)

    start_in(0, 0)
    start_in(1, 1)

    def pair_body(k, carry):
        even = 2 * k
        wait_in(0)
        lax.cond(k > 0, lambda: wait_out(0), lambda: None)
        compute(0, even)
        lax.cond(k < _NPAIR - 1, lambda: start_in(even + 2, 0), lambda: None)
        wait_in(1)
        lax.cond(k > 0, lambda: wait_out(1), lambda: None)
        compute(1, even + 1)
        lax.cond(k < _NPAIR - 1, lambda: start_in(even + 3, 1), lambda: None)
        return carry

    lax.fori_loop(0, _NPAIR, pair_body, 0)
    wait_out(0)
    wait_out(1)


def kernel(a, b, c):
    return _median_sc(a, b, c)
